# baseline (speedup 1.0000x reference)
import dataclasses

import jax
import jax.numpy as jnp
from jax import lax
from jax.experimental import pallas as pl
from jax.experimental.pallas import tpu as pltpu
from jax.experimental.pallas import tpu_sc as plsc

N = 10000
E = 320000
D = 128

NC = 2
NS = 16
NW = NC * NS
EPW = E // NW
CH = 80
NCHUNK_H = EPW // CH
RPS = 632
RPS_LAST = N - (NS - 1) * RPS


def _sc_h_body(feat_hbm, zeros_hbm, src_hbm, dst_hbm, h_out, deg_out,
               srcv, dstv, buf_a, buf_b, hist, h_sh, sem, sem2):
    c = lax.axis_index("c")
    s = lax.axis_index("s")
    w = c * NS + s
    ones16 = jnp.ones((16,), jnp.float32)
    zeros16 = jnp.zeros((16,), jnp.float32)

    @pl.when(s < NS - 1)
    def _():
        pltpu.async_copy(zeros_hbm, h_sh.at[pl.ds(s * RPS, RPS)], sem)

    @pl.when(s == NS - 1)
    def _():
        pltpu.async_copy(zeros_hbm.at[pl.ds(0, RPS_LAST)],
                         h_sh.at[pl.ds(s * RPS, RPS_LAST)], sem)

    pltpu.async_copy(src_hbm.at[pl.ds(w * EPW, EPW)], srcv, sem)
    pltpu.async_copy(dst_hbm.at[w], dstv, sem)

    @pl.loop(0, N, step=16)
    def _(i):
        hist[pl.ds(i, 16)] = zeros16

    @pl.when(s < NS - 1)
    def _():
        pltpu.make_async_copy(zeros_hbm, h_sh.at[pl.ds(s * RPS, RPS)],
                              sem).wait()

    @pl.when(s == NS - 1)
    def _():
        pltpu.make_async_copy(zeros_hbm.at[pl.ds(0, RPS_LAST)],
                              h_sh.at[pl.ds(s * RPS, RPS_LAST)], sem).wait()

    pltpu.make_async_copy(src_hbm.at[pl.ds(w * EPW, EPW)], srcv, sem).wait()
    pltpu.make_async_copy(dst_hbm.at[w], dstv, sem).wait()

    plsc.subcore_barrier()

    def _start(j, buf, sem_):
        pltpu.async_copy(feat_hbm.at[srcv.at[pl.ds(j * CH, CH)]], buf, sem_)

    def _wait(j, buf, sem_):
        pltpu.make_async_copy(feat_hbm.at[srcv.at[pl.ds(j * CH, CH)]],
                              buf, sem_).wait()

    def _scat_start(j, buf):
        pltpu.async_copy(buf, h_sh.at[dstv.at[j]], sem2, add=True)

    def _scat_wait(j, buf):
        pltpu.make_async_copy(buf, h_sh.at[dstv.at[j]], sem2).wait()

    def _count(j):
        for k in range(0, CH, 16):
            plsc.addupdate_scatter(hist, [dstv[j, pl.ds(k, 16)]], ones16)

    _start(0, buf_a, sem)
    _start(1, buf_b, sem)

    @pl.loop(0, NCHUNK_H - 3, step=2)
    def _(j):
        _count(j)
        _wait(j, buf_a, sem)
        _scat_start(j, buf_a)
        _count(j + 1)
        _wait(j + 1, buf_b, sem)
        _scat_start(j + 1, buf_b)
        _scat_wait(j, buf_a)
        _start(j + 2, buf_a, sem)
        _scat_wait(j + 1, buf_b)
        _start(j + 3, buf_b, sem)

    _count(NCHUNK_H - 3)
    _wait(NCHUNK_H - 3, buf_a, sem)
    _scat_start(NCHUNK_H - 3, buf_a)
    _count(NCHUNK_H - 2)
    _wait(NCHUNK_H - 2, buf_b, sem)
    _scat_start(NCHUNK_H - 2, buf_b)
    _scat_wait(NCHUNK_H - 3, buf_a)
    _start(NCHUNK_H - 1, buf_a, sem)
    _scat_wait(NCHUNK_H - 2, buf_b)
    _count(NCHUNK_H - 1)
    _wait(NCHUNK_H - 1, buf_a, sem)
    _scat_start(NCHUNK_H - 1, buf_a)
    _scat_wait(NCHUNK_H - 1, buf_a)

    plsc.subcore_barrier()

    base = s * RPS

    @pl.when(s < NS - 1)
    def _():
        pltpu.sync_copy(h_sh.at[pl.ds(base, RPS)],
                        h_out.at[pl.ds(c * N + base, RPS)])

    @pl.when(s == NS - 1)
    def _():
        pltpu.sync_copy(h_sh.at[pl.ds(base, RPS_LAST)],
                        h_out.at[pl.ds(c * N + base, RPS_LAST)])

    pltpu.sync_copy(hist, deg_out.at[pl.ds(w * N, N)])


_SC_CACHE = {}


def _get_sc_kernel():
    if "h" not in _SC_CACHE:
        mesh = plsc.VectorSubcoreMesh(
            core_axis_name="c", subcore_axis_name="s",
            num_cores=NC, num_subcores=NS,
        )
        cp = pltpu.CompilerParams(use_tc_tiling_on_sc=False)
        if "needs_layout_passes" in pltpu.CompilerParams.__dataclass_fields__:
            cp = dataclasses.replace(cp, needs_layout_passes=False)
        _SC_CACHE["h"] = pl.kernel(
            _sc_h_body,
            out_type=[
                jax.ShapeDtypeStruct((NC * N, D), jnp.float32),
                jax.ShapeDtypeStruct((NW * N,), jnp.float32),
            ],
            mesh=mesh,
            compiler_params=cp,
            scratch_types=[
                pltpu.VMEM((EPW,), jnp.int32),
                pltpu.VMEM((NCHUNK_H, CH), jnp.int32),
                pltpu.VMEM((CH, D), jnp.float32),
                pltpu.VMEM((CH, D), jnp.float32),
                pltpu.VMEM((N,), jnp.float32),
                pltpu.VMEM_SHARED((N, D), jnp.float32),
                pltpu.SemaphoreType.DMA,
                pltpu.SemaphoreType.DMA,
            ],
        )
    return _SC_CACHE["h"]


BLK = 2000


def _tc_body(feat_ref, h_ref, d_ref, w1_ref, w2_ref, b_ref, o_ref):
    x = feat_ref[...]
    h = h_ref[0] + h_ref[1]
    deg = d_ref[...]
    ah = h / jnp.maximum(deg, 1.0)
    acc = lax.dot_general(x, w1_ref[...], (((1,), (1,)), ((), ())),
                          precision=lax.Precision.HIGHEST,
                          preferred_element_type=jnp.float32)
    acc = acc + lax.dot_general(ah, w2_ref[...], (((1,), (1,)), ((), ())),
                                precision=lax.Precision.HIGHEST,
                                preferred_element_type=jnp.float32)
    o_ref[...] = acc + b_ref[...]


_tc_linear = pl.pallas_call(
    _tc_body,
    grid=(N // BLK,),
    in_specs=[
        pl.BlockSpec((BLK, D), lambda i: (i, 0)),
        pl.BlockSpec((NC, BLK, D), lambda i: (0, i, 0)),
        pl.BlockSpec((BLK, 1), lambda i: (i, 0)),
        pl.BlockSpec((D, D), lambda i: (0, 0)),
        pl.BlockSpec((D, D), lambda i: (0, 0)),
        pl.BlockSpec((1, D), lambda i: (0, 0)),
    ],
    out_specs=pl.BlockSpec((BLK, D), lambda i: (i, 0)),
    out_shape=jax.ShapeDtypeStruct((N, D), jnp.float32),
)


@jax.jit
def kernel(feat, edge_index, W1, b1, W2, b2):
    src = edge_index[0].astype(jnp.int32)
    dst_h = edge_index[1].astype(jnp.int32).reshape(NW, NCHUNK_H, CH)
    sc_h = _get_sc_kernel()
    zeros_blk = jnp.zeros((RPS, D), jnp.float32)
    h_flat, deg_flat = sc_h(feat, zeros_blk, src, dst_h)
    h_parts = h_flat.reshape(NC, N, D)
    deg_w = deg_flat.reshape(NW, N).sum(axis=0).reshape(N, 1)
    bias = (b1 + b2).reshape(1, D)
    return _tc_linear(feat, h_parts, deg_w, W1, W2, bias)


# speedup vs baseline: 1.0169x; 1.0169x over previous
import dataclasses

import jax
import jax.numpy as jnp
from jax import lax
from jax.experimental import pallas as pl
from jax.experimental.pallas import tpu as pltpu
from jax.experimental.pallas import tpu_sc as plsc

N = 10000
E = 320000
D = 128

NC = 2
NS = 16
NW = NC * NS
EPW = E // NW
CH = 80
NCHUNK_H = EPW // CH
RPS = 632
RPS_LAST = N - (NS - 1) * RPS


def _sc_h_body(feat_hbm, zeros_hbm, src_hbm, dst_hbm, h_out, deg_out,
               srcv, dstv, buf_a, buf_b, hist, h_sh, sem):
    c = lax.axis_index("c")
    s = lax.axis_index("s")
    w = c * NS + s
    ones16 = jnp.ones((16,), jnp.float32)
    zeros16 = jnp.zeros((16,), jnp.float32)

    @pl.when(s < NS - 1)
    def _():
        pltpu.async_copy(zeros_hbm, h_sh.at[pl.ds(s * RPS, RPS)], sem)

    @pl.when(s == NS - 1)
    def _():
        pltpu.async_copy(zeros_hbm.at[pl.ds(0, RPS_LAST)],
                         h_sh.at[pl.ds(s * RPS, RPS_LAST)], sem)

    pltpu.async_copy(src_hbm.at[pl.ds(w * EPW, EPW)], srcv, sem)
    pltpu.async_copy(dst_hbm.at[w], dstv, sem)

    @pl.loop(0, N, step=16)
    def _(i):
        hist[pl.ds(i, 16)] = zeros16

    @pl.when(s < NS - 1)
    def _():
        pltpu.make_async_copy(zeros_hbm, h_sh.at[pl.ds(s * RPS, RPS)],
                              sem).wait()

    @pl.when(s == NS - 1)
    def _():
        pltpu.make_async_copy(zeros_hbm.at[pl.ds(0, RPS_LAST)],
                              h_sh.at[pl.ds(s * RPS, RPS_LAST)], sem).wait()

    pltpu.make_async_copy(src_hbm.at[pl.ds(w * EPW, EPW)], srcv, sem).wait()
    pltpu.make_async_copy(dst_hbm.at[w], dstv, sem).wait()

    plsc.subcore_barrier()

    def _start(j, buf, sem_):
        pltpu.async_copy(feat_hbm.at[srcv.at[pl.ds(j * CH, CH)]], buf, sem_)

    def _wait(j, buf, sem_):
        pltpu.make_async_copy(feat_hbm.at[srcv.at[pl.ds(j * CH, CH)]],
                              buf, sem_).wait()

    def _scatter(j, buf):
        pltpu.sync_copy(buf, h_sh.at[dstv.at[j]], add=True)

    def _count(j):
        for k in range(0, CH, 16):
            plsc.addupdate_scatter(hist, [dstv[j, pl.ds(k, 16)]], ones16)

    _start(0, buf_a, sem)
    _start(1, buf_b, sem)

    @pl.loop(0, NCHUNK_H - 3, step=2)
    def _(j):
        _count(j)
        _wait(j, buf_a, sem)
        _scatter(j, buf_a)
        _start(j + 2, buf_a, sem)
        _count(j + 1)
        _wait(j + 1, buf_b, sem)
        _scatter(j + 1, buf_b)
        _start(j + 3, buf_b, sem)

    _count(NCHUNK_H - 3)
    _wait(NCHUNK_H - 3, buf_a, sem)
    _scatter(NCHUNK_H - 3, buf_a)
    _start(NCHUNK_H - 1, buf_a, sem)
    _count(NCHUNK_H - 2)
    _wait(NCHUNK_H - 2, buf_b, sem)
    _scatter(NCHUNK_H - 2, buf_b)
    _count(NCHUNK_H - 1)
    _wait(NCHUNK_H - 1, buf_a, sem)
    _scatter(NCHUNK_H - 1, buf_a)

    plsc.subcore_barrier()

    base = s * RPS

    @pl.when(s < NS - 1)
    def _():
        pltpu.sync_copy(h_sh.at[pl.ds(base, RPS)],
                        h_out.at[pl.ds(c * N + base, RPS)])

    @pl.when(s == NS - 1)
    def _():
        pltpu.sync_copy(h_sh.at[pl.ds(base, RPS_LAST)],
                        h_out.at[pl.ds(c * N + base, RPS_LAST)])

    pltpu.sync_copy(hist, deg_out.at[pl.ds(w * N, N)])


_SC_CACHE = {}


def _get_sc_kernel():
    if "h" not in _SC_CACHE:
        mesh = plsc.VectorSubcoreMesh(
            core_axis_name="c", subcore_axis_name="s",
            num_cores=NC, num_subcores=NS,
        )
        cp = pltpu.CompilerParams(use_tc_tiling_on_sc=False)
        if "needs_layout_passes" in pltpu.CompilerParams.__dataclass_fields__:
            cp = dataclasses.replace(cp, needs_layout_passes=False)
        _SC_CACHE["h"] = pl.kernel(
            _sc_h_body,
            out_type=[
                jax.ShapeDtypeStruct((NC * N, D), jnp.float32),
                jax.ShapeDtypeStruct((NW * N,), jnp.float32),
            ],
            mesh=mesh,
            compiler_params=cp,
            scratch_types=[
                pltpu.VMEM((EPW,), jnp.int32),
                pltpu.VMEM((NCHUNK_H, CH), jnp.int32),
                pltpu.VMEM((CH, D), jnp.float32),
                pltpu.VMEM((CH, D), jnp.float32),
                pltpu.VMEM((N,), jnp.float32),
                pltpu.VMEM_SHARED((N, D), jnp.float32),
                pltpu.SemaphoreType.DMA,
            ],
        )
    return _SC_CACHE["h"]


BLK = 2000


def _tc_self_body(feat_ref, w1_ref, b_ref, o_ref):
    o_ref[...] = lax.dot_general(feat_ref[...], w1_ref[...],
                                 (((1,), (1,)), ((), ())),
                                 precision=lax.Precision.HIGHEST,
                                 preferred_element_type=jnp.float32) + b_ref[...]


_tc_self = pl.pallas_call(
    _tc_self_body,
    grid=(N // BLK,),
    in_specs=[
        pl.BlockSpec((BLK, D), lambda i: (i, 0)),
        pl.BlockSpec((D, D), lambda i: (0, 0)),
        pl.BlockSpec((1, D), lambda i: (0, 0)),
    ],
    out_specs=pl.BlockSpec((BLK, D), lambda i: (i, 0)),
    out_shape=jax.ShapeDtypeStruct((N, D), jnp.float32),
)


def _tc_agg_body(tmp_ref, h_ref, d_ref, w2_ref, o_ref):
    h = h_ref[0] + h_ref[1]
    deg = d_ref[...]
    ah = h / jnp.maximum(deg, 1.0)
    o_ref[...] = tmp_ref[...] + lax.dot_general(
        ah, w2_ref[...], (((1,), (1,)), ((), ())),
        precision=lax.Precision.HIGHEST,
        preferred_element_type=jnp.float32)


_tc_agg = pl.pallas_call(
    _tc_agg_body,
    grid=(N // BLK,),
    in_specs=[
        pl.BlockSpec((BLK, D), lambda i: (i, 0)),
        pl.BlockSpec((NC, BLK, D), lambda i: (0, i, 0)),
        pl.BlockSpec((BLK, 1), lambda i: (i, 0)),
        pl.BlockSpec((D, D), lambda i: (0, 0)),
    ],
    out_specs=pl.BlockSpec((BLK, D), lambda i: (i, 0)),
    out_shape=jax.ShapeDtypeStruct((N, D), jnp.float32),
)


@jax.jit
def kernel(feat, edge_index, W1, b1, W2, b2):
    src = edge_index[0].astype(jnp.int32)
    dst_h = edge_index[1].astype(jnp.int32).reshape(NW, NCHUNK_H, CH)
    sc_h = _get_sc_kernel()
    zeros_blk = jnp.zeros((RPS, D), jnp.float32)
    h_flat, deg_flat = sc_h(feat, zeros_blk, src, dst_h)
    h_parts = h_flat.reshape(NC, N, D)
    deg_w = deg_flat.reshape(NW, N).sum(axis=0).reshape(N, 1)
    bias = (b1 + b2).reshape(1, D)
    tmp = _tc_self(feat, W1, bias)
    return _tc_agg(tmp, h_parts, deg_w, W2)
